# Initial kernel scaffold; baseline (speedup 1.0000x reference)
import jax
import jax.numpy as jnp
from jax.experimental import pallas as pl


def kernel(params, nodes_u, nodes_v, history_u, history_ur, history_v, history_vr, social_adj, call_training):
    raise NotImplementedError("write your pallas kernel here")


# speedup vs baseline: 2.0532x; 2.0532x over previous
import functools

import jax
import jax.numpy as jnp
from jax import lax
from jax.experimental import pallas as pl
from jax.experimental.pallas import tpu as pltpu
from jax.experimental.pallas import tpu_sc as plsc

D = 64
H = 20
S = 20
NW = 32


def _dot(a, b):
    return lax.dot(a, b, preferred_element_type=jnp.float32)


def _dot_hi(a, b):
    return lax.dot(a, b, precision=lax.Precision.HIGHEST,
                   preferred_element_type=jnp.float32)


def _bf(x):
    return x.astype(jnp.bfloat16).astype(jnp.float32)


def _uv_block(euv_ref, ridx_ref, urep_ref, r2e_ref,
              w1a_ref, w1b_ref, b1_ref, w2_ref, b2_ref,
              a1a_ref, a1b_ref, ba1_ref, a2_ref, ba2_ref, a3_ref,
              ela_ref, elb_ref, bel_ref, out_ref):
    NB, Dd = out_ref.shape
    Hh = euv_ref.shape[0] // NB
    euv = euv_ref[:, :Dd]
    oh = (ridx_ref[...] == lax.broadcasted_iota(jnp.int32, (1, 8), 1))
    oh = oh.astype(jnp.float32)
    er = _dot_hi(oh, r2e_ref[...])
    x = jnp.maximum(
        _dot(euv, w1a_ref[...]) + _dot(er, w1b_ref[...]) + b1_ref[...], 0.0)
    o = jnp.maximum(_dot(x, w2_ref[...]) + b2_ref[...], 0.0)
    urep = urep_ref[...]
    a = (_dot(o, a1a_ref[...]).reshape(NB, Hh, Dd)
         + _dot(urep, a1b_ref[...]).reshape(NB, 1, Dd)
         + ba1_ref[...].reshape(1, 1, Dd))
    a = jnp.maximum(a, 0.0).reshape(NB * Hh, Dd)
    a = jnp.maximum(_dot(a, a2_ref[...]) + ba2_ref[...], 0.0)
    ones_dd = jnp.ones((Dd, Dd), jnp.float32)
    lr = _dot_hi(_bf(a) * _bf(a3_ref[...]), ones_dd).reshape(NB, Hh, Dd)
    m = jnp.max(lr, axis=1, keepdims=True)
    e = jnp.exp(lr - m)
    w = e / jnp.sum(e, axis=1, keepdims=True)
    agg = jnp.sum(o.reshape(NB, Hh, Dd) * w, axis=1)
    out_ref[...] = jnp.maximum(
        _dot(urep, ela_ref[...]) + _dot(agg, elb_ref[...]) + bel_ref[...], 0.0)


def _uv_encode_tc(euv, ridx_col, urep, r2e_pad, p, pfx, lin_name, nb):
    n = urep.shape[0]
    assert n % nb == 0
    grid = (n // nb,)
    w1, b1 = p[pfx + 'w_r1']
    w2, b2 = p[pfx + 'w_r2']
    a1, ba1 = p[pfx + 'att1']
    a2, ba2 = p[pfx + 'att2']
    a3, _ba3 = p[pfx + 'att3']
    el, bel = p[lin_name]
    row = lambda v: v.reshape(1, -1)
    full = lambda arr: pl.BlockSpec(arr.shape, lambda i: (0, 0))
    args = (
        euv, ridx_col, urep, r2e_pad,
        w1[:D], w1[D:], row(b1), w2, row(b2),
        a1[:D], a1[D:], row(ba1), a2, row(ba2), a3.reshape(1, D),
        el[:D], el[D:], row(bel),
    )
    in_specs = [
        pl.BlockSpec((nb * H, NC), lambda i: (i, 0)),
        pl.BlockSpec((nb * H, 1), lambda i: (i, 0)),
        pl.BlockSpec((nb, D), lambda i: (i, 0)),
    ] + [full(a) for a in args[3:]]
    return pl.pallas_call(
        _uv_block,
        grid=grid,
        in_specs=in_specs,
        out_specs=pl.BlockSpec((nb, D), lambda i: (i, 0)),
        out_shape=jax.ShapeDtypeStruct((n, D), jnp.float32),
    )(*args)


def _final_block(nf_ref, selff_ref, urepb_ref, embv_ref,
                 s1a, s1b, bs1, s2w, bs2, s3w,
                 senca, sencb, bsenc,
                 wur1, bur1, g1, be1, wur2, bur2,
                 wvr1, bvr1, g2, be2, wvr2, bvr2,
                 wuv1a, wuv1b, buv1, g3, be3,
                 wuv2, buv2, g4, be4, wuv3, buv3, out_ref):
    B, Dd = urepb_ref.shape
    Ss = nf_ref.shape[0] // B
    nf = nf_ref[...]
    urep = urepb_ref[...]
    a = (_dot(nf, s1a[...]).reshape(B, Ss, Dd)
         + _dot(urep, s1b[...]).reshape(B, 1, Dd)
         + bs1[...].reshape(1, 1, Dd))
    a = jnp.maximum(a, 0.0).reshape(B * Ss, Dd)
    a = jnp.maximum(_dot(a, s2w[...]) + bs2[...], 0.0)
    ones_dd = jnp.ones((Dd, Dd), jnp.float32)
    lr = _dot_hi(_bf(a) * _bf(s3w[...]), ones_dd).reshape(B, Ss, Dd)
    m = jnp.max(lr, axis=1, keepdims=True)
    e = jnp.exp(lr - m)
    w = e / jnp.sum(e, axis=1, keepdims=True)
    agg = jnp.sum(nf.reshape(B, Ss, Dd) * w, axis=1)
    embu = jnp.maximum(
        _dot(selff_ref[...], senca[...]) + _dot(agg, sencb[...]) + bsenc[...], 0.0)

    def bn(t, g, b):
        mu = jnp.mean(t, axis=0, keepdims=True)
        var = jnp.mean((t - mu) ** 2, axis=0, keepdims=True)
        return g[...] * (t - mu) / jnp.sqrt(var + 1e-3) + b[...]

    xu = jnp.maximum(bn(_dot(embu, wur1[...]) + bur1[...], g1, be1), 0.0)
    xu = _dot(xu, wur2[...]) + bur2[...]
    xv = jnp.maximum(bn(_dot(embv_ref[...], wvr1[...]) + bvr1[...], g2, be2), 0.0)
    xv = _dot(xv, wvr2[...]) + bvr2[...]
    x = jnp.maximum(bn(_dot(xu, wuv1a[...]) + _dot(xv, wuv1b[...]) + buv1[...],
                       g3, be3), 0.0)
    x = jnp.maximum(bn(_dot(x, wuv2[...]) + buv2[...], g4, be4), 0.0)
    out_ref[...] = (jnp.sum(_bf(x) * _bf(wuv3[...]), axis=1, keepdims=True)
                    + buv3[...])


def _final_tc(nf, selff, urep_b, emb_v, p):
    B = urep_b.shape[0]
    row = lambda v: v.reshape(1, -1)
    s1, bs1 = p['satt1']
    s2, bs2 = p['satt2']
    s3, _ = p['satt3']
    senc, bsenc = p['senc_lin']
    wur1, bur1 = p['w_ur1']
    wur2, bur2 = p['w_ur2']
    wvr1, bvr1 = p['w_vr1']
    wvr2, bvr2 = p['w_vr2']
    wuv1, buv1 = p['w_uv1']
    wuv2, buv2 = p['w_uv2']
    wuv3, buv3 = p['w_uv3']
    g1, be1 = p['bn1']
    g2, be2 = p['bn2']
    g3, be3 = p['bn3']
    g4, be4 = p['bn4']
    args = (
        nf, selff, urep_b, emb_v,
        s1[:D], s1[D:], row(bs1), s2, row(bs2), s3.reshape(1, D),
        senc[:D], senc[D:], row(bsenc),
        wur1, row(bur1), row(g1), row(be1), wur2, row(bur2),
        wvr1, row(bvr1), row(g2), row(be2), wvr2, row(bvr2),
        wuv1[:D], wuv1[D:], row(buv1), row(g3), row(be3),
        wuv2, row(buv2), row(g4), row(be4), wuv3.reshape(1, 16), row(buv3),
    )
    return pl.pallas_call(
        _final_block,
        out_shape=jax.ShapeDtypeStruct((B, 1), jnp.float32),
    )(*args)


NC = 128


def _sc_mesh():
    return plsc.VectorSubcoreMesh(core_axis_name="c", subcore_axis_name="s")


def _wid():
    return lax.axis_index("s") * 2 + lax.axis_index("c")


def _fire_gathers(table_h, idx_v, buf_v, sem, nchunks, c, idx_off=0):
    cps = []
    for j in range(nchunks):
        cps.append(pltpu.async_copy(
            table_h.at[idx_v.at[pl.ds(idx_off + j * c, c)]],
            buf_v.at[pl.ds(j * c, c)], sem))
    return cps


def _sc_gather_pair(u_comb, v_comb, nodes_u, nodes_v):
    B = nodes_u.shape[0]
    n_w = B // NW
    outs = (jax.ShapeDtypeStruct((B, NC), jnp.int32),
            jax.ShapeDtypeStruct((B, NC), jnp.int32))
    scratch = [pltpu.VMEM((n_w,), jnp.int32),
               pltpu.VMEM((n_w,), jnp.int32),
               pltpu.VMEM((n_w, NC), jnp.int32),
               pltpu.VMEM((n_w, NC), jnp.int32),
               pltpu.SemaphoreType.DMA]

    @functools.partial(pl.kernel, out_type=outs, mesh=_sc_mesh(),
                       scratch_types=scratch)
    def k(uc_h, vc_h, nu_h, nv_h, au_o, av_o, idxu_v, idxv_v, bu, bv, sem):
        base = _wid() * n_w
        pltpu.sync_copy(nu_h.at[pl.ds(base, n_w)], idxu_v)
        pltpu.sync_copy(nv_h.at[pl.ds(base, n_w)], idxv_v)
        cps = [pltpu.async_copy(uc_h.at[idxu_v], bu, sem),
               pltpu.async_copy(vc_h.at[idxv_v], bv, sem)]
        for cp in cps:
            cp.wait()
        pltpu.sync_copy(bu, au_o.at[pl.ds(base, n_w)])
        pltpu.sync_copy(bv, av_o.at[pl.ds(base, n_w)])

    return k(u_comb, v_comb, nodes_u, nodes_v)


def _sc_gather_table(table, ids):
    N = ids.shape[0]
    n_w = N // NW
    c = 128
    nch = n_w // c
    outs = jax.ShapeDtypeStruct((N, NC), jnp.int32)
    scratch = [pltpu.VMEM((n_w,), jnp.int32),
               pltpu.VMEM((n_w, NC), jnp.int32),
               pltpu.SemaphoreType.DMA]

    @functools.partial(pl.kernel, out_type=outs, mesh=_sc_mesh(),
                       scratch_types=scratch)
    def k(t_h, ids_h, out_o, idx_v, buf, sem):
        base = _wid() * n_w
        pltpu.sync_copy(ids_h.at[pl.ds(base, n_w)], idx_v)
        cps = _fire_gathers(t_h, idx_v, buf, sem, nch, c)
        for cp in cps:
            cp.wait()
        pltpu.sync_copy(buf, out_o.at[pl.ds(base, n_w)])

    return k(table, ids)


def _sc_gather_embeds(v_comb, u_comb, ids_u, ids_v):
    NU, NV = ids_u.shape[0], ids_v.shape[0]
    n_wu, n_wv = NU // NW, NV // NW
    c, k1 = 128, 5
    sup = c * k1
    nsup = n_wu // sup
    nchv = n_wv // c
    outs = (jax.ShapeDtypeStruct((NU, NC), jnp.int32),
            jax.ShapeDtypeStruct((NV, NC), jnp.int32))
    scratch = [pltpu.VMEM((n_wu,), jnp.int32),
               pltpu.VMEM((n_wv,), jnp.int32),
               pltpu.VMEM((sup, NC), jnp.int32),
               pltpu.SemaphoreType.DMA]

    @functools.partial(pl.kernel, out_type=outs, mesh=_sc_mesh(),
                       scratch_types=scratch)
    def k(vc_h, uc_h, idu_h, idv_h, euv_o, evu_o, idxu_v, idxv_v, buf, sem):
        bu = _wid() * n_wu
        bv = _wid() * n_wv
        pltpu.sync_copy(idu_h.at[pl.ds(bu, n_wu)], idxu_v)
        pltpu.sync_copy(idv_h.at[pl.ds(bv, n_wv)], idxv_v)

        def body(s, carry):
            off = s * sup
            cps = [pltpu.async_copy(
                vc_h.at[idxu_v.at[pl.ds(off + j * c, c)]],
                buf.at[pl.ds(j * c, c)], sem) for j in range(k1)]
            for cp in cps:
                cp.wait()
            pltpu.sync_copy(buf, euv_o.at[pl.ds(bu + off, sup)])
            return carry

        lax.fori_loop(0, nsup, body, 0)
        cps = _fire_gathers(uc_h, idxv_v, buf, sem, nchv, c)
        for cp in cps:
            cp.wait()
        pltpu.sync_copy(buf, evu_o.at[pl.ds(bv, n_wv)])

    return k(v_comb, u_comb, ids_u, ids_v)


def kernel(params, nodes_u, nodes_v, history_u, history_ur, history_v,
           history_vr, social_adj, call_training=False):
    p = params
    B = nodes_u.shape[0]
    r2e_pad = jnp.zeros((8, D), jnp.float32).at[:5].set(p['r2e'])

    nu_rows = p['u2e'].shape[0]
    nv_rows = p['v2e'].shape[0]
    u2e_i = lax.bitcast_convert_type(p['u2e'], jnp.int32)
    v2e_i = lax.bitcast_convert_type(p['v2e'], jnp.int32)
    u_comb = jnp.concatenate(
        [u2e_i, history_u.astype(jnp.int32), history_ur.astype(jnp.int32),
         social_adj.astype(jnp.int32),
         jnp.zeros((nu_rows, NC - D - H - H - S), jnp.int32)], axis=1)
    v_comb = jnp.concatenate(
        [v2e_i, history_v.astype(jnp.int32), history_vr.astype(jnp.int32),
         jnp.zeros((nv_rows, NC - D - H - H), jnp.int32)], axis=1)

    au, av = _sc_gather_pair(u_comb, v_comb, nodes_u, nodes_v)
    neigh = au[:, D + 2 * H:D + 2 * H + S]
    hist_b = au[:, D:D + H]
    rhist_b = au[:, D + H:D + 2 * H]
    u2e_b = lax.bitcast_convert_type(au[:, :D], jnp.float32)
    hist_v = av[:, D:D + H]
    rhist_v = av[:, D + H:D + 2 * H]
    vrep = lax.bitcast_convert_type(av[:, :D], jnp.float32)

    an = _sc_gather_table(u_comb, neigh.reshape(-1))
    hist_n = an[:, D:D + H]
    rhist_n = an[:, D + H:D + 2 * H]
    urep_n = lax.bitcast_convert_type(an[:, :D], jnp.float32)

    euv_ids = jnp.concatenate([hist_n, hist_b]).reshape(-1)
    rhist_u = jnp.concatenate([rhist_n, rhist_b])
    urep_u = jnp.concatenate([urep_n, u2e_b])

    euv_c, evu_c = _sc_gather_embeds(
        v_comb, u_comb, euv_ids, hist_v.reshape(-1))
    euv = lax.bitcast_convert_type(euv_c, jnp.float32)
    evu = lax.bitcast_convert_type(evu_c, jnp.float32)

    feats_u = _uv_encode_tc(euv, rhist_u.reshape(-1, 1), urep_u, r2e_pad,
                            p, 'aggu_', 'encu_lin', nb=128)
    emb_v = _uv_encode_tc(evu, rhist_v.reshape(-1, 1), vrep, r2e_pad,
                          p, 'aggv_', 'encv_lin', nb=128)
    nf = feats_u[:B * S]
    selff = feats_u[B * S:]
    urep_b = urep_u[B * S:]
    scores = _final_tc(nf, selff, urep_b, emb_v, p)
    return scores[:, 0]
